# baseline (speedup 1.0000x reference)
import functools

import jax
import jax.numpy as jnp
import numpy as np
from jax import lax
from jax.experimental import pallas as pl
from jax.experimental.pallas import tpu as pltpu
from jax.experimental.pallas import tpu_sc as plsc

NC, NS, L = 2, 16, 16
NW = NC * NS
HASH_BITS = 11
S = 1 << HASH_BITS
MULT = np.int32(-1640531527)
LPC = 5


def _hash(v):
    return lax.shift_right_logical(v * MULT, 32 - HASH_BITS)


def _splat(v):
    return jnp.full((L,), v, jnp.int32)


def _sload(ref, i):
    return jnp.max(plsc.load_gather(ref, [_splat(i)]))


def _sstore(ref, i, v):
    lane0 = lax.broadcasted_iota(jnp.int32, (L,), 0) == 0
    plsc.store_scatter(ref, [_splat(i)], _splat(v), mask=lane0)


def _lane(vec, lane, fill):
    lanes = lax.broadcasted_iota(jnp.int32, (L,), 0)
    return jnp.max(jnp.where(lanes == lane, vec, np.int32(fill)))


def _sc_body(ntok, seq, bpw, x_hbm, w_hbm, tok_hbm, delta_hbm, out_hbm,
             idx_v, idx2, rowbuf, tbuf, tokv, keys, vals, dloc, sem):
    wid = lax.axis_index("s") * NC + lax.axis_index("c")
    base = pl.multiple_of(wid * bpw * seq, 128)
    b0 = pl.multiple_of(wid * bpw, 128)
    lanes = lax.broadcasted_iota(jnp.int32, (L,), 0)

    pltpu.sync_copy(x_hbm.at[pl.ds(base, bpw * seq)], idx_v)
    pltpu.sync_copy(tok_hbm, tokv)
    pltpu.sync_copy(delta_hbm, dloc)

    def init(i, _):
        keys[pl.ds(pl.multiple_of(i * L, L), L)] = _splat(np.int32(-1))
        return 0
    lax.fori_loop(0, S // L, init, 0)

    def insert(k, _):
        t = _sload(tokv, k)
        h0 = _hash(t)
        kh0 = _sload(keys, h0)

        def cond(st):
            _, kh = st
            return (kh != -1) & (kh != t)

        def body(st):
            h, _ = st
            h2 = (h + 1) & (S - 1)
            return h2, _sload(keys, h2)

        h, _ = lax.while_loop(cond, body, (h0, kh0))
        _sstore(keys, h, t)
        _sstore(vals, h, k)
        return 0
    lax.fori_loop(0, ntok, insert, 0)

    def chunk_body(ch, _):
        l0 = ch * LPC

        def mkidx(v, _):
            li = v // (bpw // L)
            j = v % (bpw // L)
            pos = (j * L + lanes) * seq + l0 + li
            idx2[li, pl.ds(j * L, L)] = plsc.load_gather(idx_v, [pos])
            return 0
        lax.fori_loop(0, LPC * (bpw // L), mkidx, 0)

        copies = []
        for li in range(LPC):
            copies.append(pltpu.async_copy(
                w_hbm.at[idx2.at[li]],
                rowbuf.at[pl.ds(li * 128, 128)], sem))
        for cp in copies:
            cp.wait()

        def fix(v, _):
            xv = idx2[v // (bpw // L), pl.ds((v % (bpw // L)) * L, L)]
            hv = _hash(xv)
            kh0 = plsc.load_gather(keys, [hv])

            def vcond(st):
                _, kh = st
                alive = (kh != -1) & (kh != xv)
                return jnp.max(jnp.where(alive, 1, 0)) > 0

            def vbody(st):
                hv_, kh = st
                alive = (kh != -1) & (kh != xv)
                hv2 = jnp.where(alive, (hv_ + 1) & (S - 1), hv_)
                return hv2, plsc.load_gather(keys, [hv2])

            hv, kh = lax.while_loop(vcond, vbody, (hv, kh0))
            found = kh == xv

            @pl.when(jnp.max(jnp.where(found, 1, 0)) > 0)
            def _():
                kk = jnp.where(found, plsc.load_gather(vals, [hv]),
                               np.int32(-1))
                for lane in range(L):
                    klane = _lane(kk, lane, -1)

                    @pl.when(klane >= 0)
                    def _():
                        row = v * L + lane
                        for c in range(64 // L):
                            rowbuf[row, pl.ds(c * L, L)] = \
                                dloc[klane, pl.ds(c * L, L)]
            return 0
        lax.fori_loop(0, LPC * (bpw // L), fix, 0)

        for li in range(LPC):
            def tr(e, _):
                for cb in range(bpw // L):
                    vals_ = plsc.load_gather(
                        rowbuf, [li * 128 + cb * L + lanes, _splat(e)])
                    tbuf[e, pl.ds(cb * L, L)] = vals_
                return 0
            lax.fori_loop(0, 64, tr, 0)
            pltpu.sync_copy(tbuf, out_hbm.at[l0 + li, :, pl.ds(b0, bpw)])
        return 0
    lax.fori_loop(0, seq // LPC, chunk_body, 0)


def kernel(x, W, token_indices, delta):
    b, l = x.shape
    vocab, embed = W.shape
    ntok = token_indices.shape[0]
    total = b * l
    bpw = b // NW
    assert b % (NW * 128) == 0 and embed == 64 and l % LPC == 0

    xf = x.reshape(total).astype(jnp.int32)
    tok = token_indices.astype(jnp.int32)

    mesh = plsc.VectorSubcoreMesh(core_axis_name="c", subcore_axis_name="s",
                                  num_cores=NC, num_subcores=NS)
    run = pl.kernel(
        functools.partial(_sc_body, ntok, l, bpw),
        out_type=jax.ShapeDtypeStruct((l, embed, b), jnp.float32),
        mesh=mesh,
        scratch_types=[
            pltpu.VMEM((bpw * l,), jnp.int32),
            pltpu.VMEM((LPC, 128), jnp.int32),
            pltpu.VMEM((LPC * 128, embed), jnp.float32),
            pltpu.VMEM((embed, bpw), jnp.float32),
            pltpu.VMEM((ntok,), jnp.int32),
            pltpu.VMEM((S,), jnp.int32),
            pltpu.VMEM((S,), jnp.int32),
            pltpu.VMEM((ntok, embed), jnp.float32),
            pltpu.SemaphoreType.DMA,
        ],
        compiler_params=pltpu.CompilerParams(needs_layout_passes=False,
                                             use_tc_tiling_on_sc=False),
    )
    out3 = run(xf, W, tok, delta)
    return jnp.transpose(out3, (2, 0, 1))
